# Initial kernel scaffold; baseline (speedup 1.0000x reference)
import jax
import jax.numpy as jnp
from jax.experimental import pallas as pl


def kernel(x, table):
    raise NotImplementedError("write your pallas kernel here")


# speedup vs baseline: 2.7651x; 2.7651x over previous
import functools

import jax
import jax.numpy as jnp
from jax import lax
from jax.experimental import pallas as pl
from jax.experimental.pallas import tpu as pltpu
from jax.experimental.pallas import tpu_sc as plsc

_D = 128
_CHUNK = 128


@functools.lru_cache(maxsize=None)
def _make_gather(n_total):
    info = plsc.get_sparse_core_info()
    nc, ns = info.num_cores, info.num_subcores
    nw = nc * ns
    n_per_w = n_total // nw
    assert n_per_w * nw == n_total and n_per_w % _CHUNK == 0
    n_chunks = n_per_w // _CHUNK
    mesh = plsc.VectorSubcoreMesh(core_axis_name="c", subcore_axis_name="s")

    @functools.partial(
        pl.kernel,
        mesh=mesh,
        out_type=jax.ShapeDtypeStruct((n_total, _D), jnp.float32),
        scratch_types=[
            pltpu.VMEM((_CHUNK,), jnp.int32),
            pltpu.VMEM((_CHUNK, _D), jnp.float32),
            pltpu.SemaphoreType.DMA,
        ],
    )
    def k(idx_hbm, table_hbm, out_hbm, idx_v, rows_v, sem):
        wid = lax.axis_index("s") * nc + lax.axis_index("c")
        base = wid * n_per_w

        def body(i, carry):
            off = base + i * _CHUNK
            pltpu.sync_copy(idx_hbm.at[pl.ds(off, _CHUNK)], idx_v)
            pltpu.async_copy(table_hbm.at[idx_v], rows_v, sem).wait()
            pltpu.sync_copy(rows_v, out_hbm.at[pl.ds(off, _CHUNK)])
            return carry

        lax.fori_loop(0, n_chunks, body, 0)

    return k


def kernel(x, table):
    b, h = x.shape
    n = b * h
    flat = x.reshape(n).astype(jnp.int32)
    out = _make_gather(n)(flat, table)
    return out.reshape(b, h, _D)


# speedup vs baseline: 3.3361x; 1.2065x over previous
import functools

import jax
import jax.numpy as jnp
from jax import lax
from jax.experimental import pallas as pl
from jax.experimental.pallas import tpu as pltpu
from jax.experimental.pallas import tpu_sc as plsc

_D = 128
_CHUNK = 128
_NBUF = 5
_PF = 2


@functools.lru_cache(maxsize=None)
def _make_gather(nw, nc, n_chunks):
    n_total = nw * n_chunks * _CHUNK
    mesh = plsc.VectorSubcoreMesh(core_axis_name="c", subcore_axis_name="s")

    @functools.partial(
        pl.kernel,
        mesh=mesh,
        out_type=jax.ShapeDtypeStruct((n_total, _D), jnp.float32),
        scratch_types=[
            pltpu.VMEM((n_chunks, _CHUNK), jnp.int32),
            pltpu.VMEM((_NBUF, _CHUNK, _D), jnp.float32),
        ]
        + [pltpu.SemaphoreType.DMA] * (2 * _NBUF),
    )
    def k(idx_hbm, table_hbm, out_hbm, idx_v, rows_v, *sems):
        gsem, ssem = sems[:_NBUF], sems[_NBUF:]
        wid = lax.axis_index("s") * nc + lax.axis_index("c")
        base = wid * n_chunks

        def start_gather(c, b):
            pltpu.async_copy(table_hbm.at[idx_v.at[c]], rows_v.at[b], gsem[b])

        def wait_gather(b):
            pltpu.make_async_copy(
                table_hbm.at[idx_v.at[0]], rows_v.at[b], gsem[b]
            ).wait()

        def start_store(c, b):
            pltpu.async_copy(
                rows_v.at[b],
                out_hbm.at[pl.ds((base + c) * _CHUNK, _CHUNK)],
                ssem[b],
            )

        def wait_store(b):
            pltpu.make_async_copy(
                rows_v.at[b],
                out_hbm.at[pl.ds(base * _CHUNK, _CHUNK)],
                ssem[b],
            ).wait()

        pltpu.sync_copy(idx_hbm.at[wid], idx_v)

        for c in range(_PF):
            start_gather(c, c % _NBUF)

        def consume(c, b, prime, store_wait):
            if prime:
                bp = (b + _PF) % _NBUF
                if store_wait:
                    wait_store(bp)
                start_gather(c + _PF, bp)
            wait_gather(b)
            start_store(c, b)

        for c in range(_NBUF):
            consume(c, c % _NBUF, prime=True, store_wait=(c + _PF >= _NBUF))

        n_outer = n_chunks // _NBUF

        def outer(g, carry):
            for b in range(_NBUF):
                consume(g * _NBUF + b, b, prime=True, store_wait=True)
            return carry

        lax.fori_loop(1, n_outer - 1, outer, 0)

        for c in range(n_chunks - _NBUF, n_chunks):
            consume(c, c % _NBUF, prime=(c + _PF < n_chunks), store_wait=True)

        for b in range(_NBUF):
            wait_store(b)

    return k


def kernel(x, table):
    b, h = x.shape
    n = b * h
    info = plsc.get_sparse_core_info()
    nc, ns = info.num_cores, info.num_subcores
    nw = nc * ns
    n_chunks = n // (nw * _CHUNK)
    assert n_chunks * nw * _CHUNK == n
    idx = x.reshape(nw, n_chunks, _CHUNK).astype(jnp.int32)
    out = _make_gather(nw, nc, n_chunks)(idx, table)
    return out.reshape(b, h, _D)


# speedup vs baseline: 3.3444x; 1.0025x over previous
import functools

import jax
import jax.numpy as jnp
from jax import lax
from jax.experimental import pallas as pl
from jax.experimental.pallas import tpu as pltpu
from jax.experimental.pallas import tpu_sc as plsc

_D = 128
_CHUNK = 128
_NBUF = 5
_PF = 3


@functools.lru_cache(maxsize=None)
def _make_gather(nw, nc, n_chunks):
    n_total = nw * n_chunks * _CHUNK
    mesh = plsc.VectorSubcoreMesh(core_axis_name="c", subcore_axis_name="s")

    @functools.partial(
        pl.kernel,
        mesh=mesh,
        out_type=jax.ShapeDtypeStruct((n_total, _D), jnp.float32),
        scratch_types=[
            pltpu.VMEM((n_chunks, _CHUNK), jnp.int32),
            pltpu.VMEM((_NBUF, _CHUNK, _D), jnp.float32),
        ]
        + [pltpu.SemaphoreType.DMA] * (2 * _NBUF),
    )
    def k(idx_hbm, table_hbm, out_hbm, idx_v, rows_v, *sems):
        gsem, ssem = sems[:_NBUF], sems[_NBUF:]
        wid = lax.axis_index("s") * nc + lax.axis_index("c")
        base = wid * n_chunks

        def start_gather(c, b):
            pltpu.async_copy(table_hbm.at[idx_v.at[c]], rows_v.at[b], gsem[b])

        def wait_gather(b):
            pltpu.make_async_copy(
                table_hbm.at[idx_v.at[0]], rows_v.at[b], gsem[b]
            ).wait()

        def start_store(c, b):
            pltpu.async_copy(
                rows_v.at[b],
                out_hbm.at[pl.ds((base + c) * _CHUNK, _CHUNK)],
                ssem[b],
            )

        def wait_store(b):
            pltpu.make_async_copy(
                rows_v.at[b],
                out_hbm.at[pl.ds(base * _CHUNK, _CHUNK)],
                ssem[b],
            ).wait()

        pltpu.sync_copy(idx_hbm.at[wid], idx_v)

        for c in range(_PF):
            start_gather(c, c % _NBUF)

        def consume(c, b, prime, store_wait):
            if prime:
                bp = (b + _PF) % _NBUF
                if store_wait:
                    wait_store(bp)
                start_gather(c + _PF, bp)
            wait_gather(b)
            start_store(c, b)

        for c in range(_NBUF):
            consume(c, c % _NBUF, prime=True, store_wait=(c + _PF >= _NBUF))

        n_outer = n_chunks // _NBUF

        def outer(g, carry):
            for b in range(_NBUF):
                consume(g * _NBUF + b, b, prime=True, store_wait=True)
            return carry

        lax.fori_loop(1, n_outer - 1, outer, 0)

        for c in range(n_chunks - _NBUF, n_chunks):
            consume(c, c % _NBUF, prime=(c + _PF < n_chunks), store_wait=True)

        for b in range(_NBUF):
            wait_store(b)

    return k


def kernel(x, table):
    b, h = x.shape
    n = b * h
    info = plsc.get_sparse_core_info()
    nc, ns = info.num_cores, info.num_subcores
    nw = nc * ns
    n_chunks = n // (nw * _CHUNK)
    assert n_chunks * nw * _CHUNK == n
    idx = x.reshape(nw, n_chunks, _CHUNK).astype(jnp.int32)
    out = _make_gather(nw, nc, n_chunks)(idx, table)
    return out.reshape(b, h, _D)


# speedup vs baseline: 5.9614x; 1.7825x over previous
import functools

import jax
import jax.numpy as jnp
from jax import lax
from jax.experimental import pallas as pl
from jax.experimental.pallas import tpu as pltpu
from jax.experimental.pallas import tpu_sc as plsc

_D = 128
_K = 4
_NBUF = 4
_PF = 2


@functools.lru_cache(maxsize=None)
def _make_gather(nw, nc, batch, hist):
    b_per_w = batch // nw
    n_grp = b_per_w // _K
    mesh = plsc.VectorSubcoreMesh(core_axis_name="c", subcore_axis_name="s")

    @functools.partial(
        pl.kernel,
        mesh=mesh,
        out_type=jax.ShapeDtypeStruct((batch, hist, _D), jnp.float32),
        scratch_types=[
            pltpu.VMEM((b_per_w, hist), jnp.int32),
            pltpu.VMEM((_NBUF, _K, hist, _D), jnp.float32),
        ]
        + [pltpu.SemaphoreType.DMA] * (2 * _NBUF),
    )
    def k(idx_hbm, table_hbm, out_hbm, idx_v, rows_v, *sems):
        gsem, ssem = sems[:_NBUF], sems[_NBUF:]
        wid = lax.axis_index("s") * nc + lax.axis_index("c")
        bat0 = wid * b_per_w

        def start_gather(c, b):
            for j in range(_K):
                pltpu.async_copy(
                    table_hbm.at[idx_v.at[c * _K + j]],
                    rows_v.at[b, j],
                    gsem[b],
                )

        def wait_gather(b):
            for j in range(_K):
                pltpu.make_async_copy(
                    table_hbm.at[idx_v.at[0]], rows_v.at[b, j], gsem[b]
                ).wait()

        def start_store(c, b):
            pltpu.async_copy(
                rows_v.at[b],
                out_hbm.at[pl.ds(bat0 + c * _K, _K)],
                ssem[b],
            )

        def wait_store(b):
            pltpu.make_async_copy(
                rows_v.at[b],
                out_hbm.at[pl.ds(bat0, _K)],
                ssem[b],
            ).wait()

        pltpu.sync_copy(idx_hbm.at[pl.ds(bat0, b_per_w)], idx_v)

        for c in range(_PF):
            start_gather(c, c % _NBUF)

        def consume(c, b, prime, store_wait):
            if prime:
                bp = (b + _PF) % _NBUF
                if store_wait:
                    wait_store(bp)
                start_gather(c + _PF, bp)
            wait_gather(b)
            start_store(c, b)

        for c in range(_NBUF):
            consume(c, c % _NBUF, prime=True, store_wait=(c + _PF >= _NBUF))

        n_outer = n_grp // _NBUF

        def outer(g, carry):
            for b in range(_NBUF):
                consume(g * _NBUF + b, b, prime=True, store_wait=True)
            return carry

        lax.fori_loop(1, n_outer - 1, outer, 0)

        for c in range(n_grp - _NBUF, n_grp):
            consume(c, c % _NBUF, prime=(c + _PF < n_grp), store_wait=True)

        for b in range(_NBUF):
            wait_store(b)

    return k


def kernel(x, table):
    batch, hist = x.shape
    info = plsc.get_sparse_core_info()
    nc, ns = info.num_cores, info.num_subcores
    nw = nc * ns
    assert batch % (nw * _K * _NBUF) == 0
    return _make_gather(nw, nc, batch, hist)(x.astype(jnp.int32), table)


# speedup vs baseline: 5.9641x; 1.0005x over previous
import functools

import jax
import jax.numpy as jnp
from jax import lax
from jax.experimental import pallas as pl
from jax.experimental.pallas import tpu as pltpu
from jax.experimental.pallas import tpu_sc as plsc

_D = 128
_K = 4
_NBUF = 4
_PF = 2


@functools.lru_cache(maxsize=None)
def _make_gather(nw, nc, batch, hist):
    b_per_w = batch // nw
    n_grp = b_per_w // _K
    mesh = plsc.VectorSubcoreMesh(core_axis_name="c", subcore_axis_name="s")

    @functools.partial(
        pl.kernel,
        mesh=mesh,
        out_type=jax.ShapeDtypeStruct((batch, hist, _D), jnp.float32),
        scratch_types=[
            pltpu.VMEM((b_per_w, hist), jnp.int32),
            pltpu.VMEM((_NBUF, _K, hist, _D), jnp.float32),
        ]
        + [pltpu.SemaphoreType.DMA] * (2 * _NBUF),
        compiler_params=pltpu.CompilerParams(use_tc_tiling_on_sc=True),
    )
    def k(idx_hbm, table_hbm, out_hbm, idx_v, rows_v, *sems):
        gsem, ssem = sems[:_NBUF], sems[_NBUF:]
        wid = lax.axis_index("s") * nc + lax.axis_index("c")
        bat0 = wid * b_per_w

        def start_gather(c, b):
            for j in range(_K):
                pltpu.async_copy(
                    table_hbm.at[idx_v.at[c * _K + j]],
                    rows_v.at[b, j],
                    gsem[b],
                )

        def wait_gather(b):
            for j in range(_K):
                pltpu.make_async_copy(
                    table_hbm.at[idx_v.at[0]], rows_v.at[b, j], gsem[b]
                ).wait()

        def start_store(c, b):
            pltpu.async_copy(
                rows_v.at[b],
                out_hbm.at[pl.ds(bat0 + c * _K, _K)],
                ssem[b],
            )

        def wait_store(b):
            pltpu.make_async_copy(
                rows_v.at[b],
                out_hbm.at[pl.ds(bat0, _K)],
                ssem[b],
            ).wait()

        pltpu.sync_copy(idx_hbm.at[pl.ds(bat0, b_per_w)], idx_v)

        for c in range(_PF):
            start_gather(c, c % _NBUF)

        def consume(c, b, prime, store_wait):
            if prime:
                bp = (b + _PF) % _NBUF
                if store_wait:
                    wait_store(bp)
                start_gather(c + _PF, bp)
            wait_gather(b)
            start_store(c, b)

        for c in range(_NBUF):
            consume(c, c % _NBUF, prime=True, store_wait=(c + _PF >= _NBUF))

        n_outer = n_grp // _NBUF

        def outer(g, carry):
            for b in range(_NBUF):
                consume(g * _NBUF + b, b, prime=True, store_wait=True)
            return carry

        lax.fori_loop(1, n_outer - 1, outer, 0)

        for c in range(n_grp - _NBUF, n_grp):
            consume(c, c % _NBUF, prime=(c + _PF < n_grp), store_wait=True)

        for b in range(_NBUF):
            wait_store(b)

    return k


def kernel(x, table):
    batch, hist = x.shape
    info = plsc.get_sparse_core_info()
    nc, ns = info.num_cores, info.num_subcores
    nw = nc * ns
    assert batch % (nw * _K * _NBUF) == 0
    return _make_gather(nw, nc, batch, hist)(x.astype(jnp.int32), table)
